# baseline (speedup 1.0000x reference)
import functools

import jax
import jax.numpy as jnp
from jax import lax
from jax.experimental import pallas as pl
from jax.experimental.pallas import tpu as pltpu
from jax.experimental.pallas import tpu_sc as plsc

_ROWSUM_BM = 256
_SC_ROWS = 2048


def _rowsum_body(a_ref, o_ref):
    o_ref[...] = jnp.sum(a_ref[...], axis=1, keepdims=True)


@functools.lru_cache(maxsize=None)
def _make_sc_rowsum(m, r_sc, row0):
    info = plsc.get_sparse_core_info()
    nc, ns, nl = info.num_cores, info.num_subcores, info.num_lanes
    nw = nc * ns
    rp = r_sc // nw
    mesh = plsc.VectorSubcoreMesh(core_axis_name="c", subcore_axis_name="s")

    @functools.partial(
        pl.kernel,
        mesh=mesh,
        out_type=jax.ShapeDtypeStruct((r_sc,), jnp.float32),
        scratch_types=[
            pltpu.VMEM((m,), jnp.float32),
            pltpu.VMEM((m,), jnp.float32),
            pltpu.VMEM((rp,), jnp.float32),
            pltpu.SemaphoreType.DMA,
            pltpu.SemaphoreType.DMA,
        ],
        compiler_params=pltpu.CompilerParams(
            use_tc_tiling_on_sc=False, needs_layout_passes=False
        ),
    )
    def sc_rowsum(a_hbm, deg_out, buf0, buf1, out_v, sem0, sem1):
        cid = lax.axis_index("c")
        sid = lax.axis_index("s")
        wid = sid * nc + cid
        r0 = row0 + wid * rp
        bufs = (buf0, buf1)
        sems = (sem0, sem1)
        zero = jnp.zeros((nl,), jnp.float32)
        lane = lax.iota(jnp.int32, nl)
        cps = {0: pltpu.async_copy(a_hbm.at[r0], bufs[0], sems[0])}
        sums = zero
        for j in range(rp):
            k = j & 1
            cps[k].wait()
            if j + 1 < rp:
                kn = (j + 1) & 1
                cps[kn] = pltpu.async_copy(
                    a_hbm.at[r0 + (j + 1)], bufs[kn], sems[kn]
                )
            buf = bufs[k]

            def body(i, accs, buf=buf):
                base = i * (4 * nl)
                return (
                    accs[0] + buf[pl.ds(base, nl)],
                    accs[1] + buf[pl.ds(base + nl, nl)],
                    accs[2] + buf[pl.ds(base + 2 * nl, nl)],
                    accs[3] + buf[pl.ds(base + 3 * nl, nl)],
                )

            a0, a1, a2, a3 = lax.fori_loop(
                0, m // (4 * nl), body, (zero, zero, zero, zero)
            )
            s = jnp.sum((a0 + a1) + (a2 + a3))
            sums = jnp.where(lane == (j % nl), s, sums)
            if (j + 1) % nl == 0:
                out_v[pl.ds((j + 1 - nl), nl)] = sums
                sums = zero
        pltpu.sync_copy(out_v, deg_out.at[pl.ds(wid * rp, rp)])

    return sc_rowsum


@functools.lru_cache(maxsize=None)
def _make_sc_gather(b, d):
    info = plsc.get_sparse_core_info()
    nc, ns = info.num_cores, info.num_subcores
    nw = nc * ns
    bw = b // nw
    mesh = plsc.VectorSubcoreMesh(core_axis_name="c", subcore_axis_name="s")

    @functools.partial(
        pl.kernel,
        mesh=mesh,
        out_type=jax.ShapeDtypeStruct((b, d), jnp.float32),
        scratch_types=[
            pltpu.VMEM((bw,), jnp.int32),
            pltpu.VMEM((bw, d), jnp.float32),
            pltpu.SemaphoreType.DMA,
        ],
        compiler_params=pltpu.CompilerParams(
            use_tc_tiling_on_sc=False, needs_layout_passes=False
        ),
    )
    def sc_gather(table_hbm, idx_hbm, emb_out, idx_v, rows_v, sem):
        cid = lax.axis_index("c")
        sid = lax.axis_index("s")
        wid = sid * nc + cid
        base = wid * bw
        pltpu.sync_copy(idx_hbm.at[pl.ds(base, bw)], idx_v)
        cp = pltpu.async_copy(table_hbm.at[idx_v], rows_v, sem)
        cp.wait()
        pltpu.sync_copy(rows_v, emb_out.at[pl.ds(base, bw)])

    return sc_gather


@functools.lru_cache(maxsize=None)
def _make_sc_degree(b):
    info = plsc.get_sparse_core_info()
    nc, ns, nl = info.num_cores, info.num_subcores, info.num_lanes
    nw = nc * ns
    bw = b // nw
    bp = 2 * bw
    mesh = plsc.VectorSubcoreMesh(core_axis_name="c", subcore_axis_name="s")

    @functools.partial(
        pl.kernel,
        mesh=mesh,
        out_type=jax.ShapeDtypeStruct((b,), jnp.float32),
        scratch_types=[
            pltpu.VMEM((bp,), jnp.int32),
            pltpu.VMEM((bp,), jnp.float32),
            pltpu.VMEM((bw,), jnp.float32),
            pltpu.VMEM((nl,), jnp.float32),
            pltpu.VMEM((ns * nl,), jnp.float32),
            pltpu.VMEM_SHARED((ns * nl,), jnp.float32),
            pltpu.SemaphoreType.DMA,
        ],
        compiler_params=pltpu.CompilerParams(
            use_tc_tiling_on_sc=False, needs_layout_passes=False
        ),
    )
    def sc_degree(idx_hbm, deg_hbm, dn_out,
                  idxp_v, degp_v, dn_v, mx_v, allmx_v, shared_mx, sem):
        cid = lax.axis_index("c")
        sid = lax.axis_index("s")
        wid = sid * nc + cid
        base = wid * bw
        pair_base = (wid // 2) * bp
        off = (wid % 2) * bw
        pltpu.sync_copy(idx_hbm.at[pl.ds(pair_base, bp)], idxp_v)
        cp = pltpu.async_copy(deg_hbm.at[idxp_v], degp_v, sem)
        cp.wait()
        mx = degp_v[pl.ds(0, nl)]
        for i in range(1, bp // nl):
            mx = jnp.maximum(mx, degp_v[pl.ds(i * nl, nl)])
        mx_v[...] = mx
        pltpu.sync_copy(mx_v, shared_mx.at[pl.ds(sid * nl, nl)])
        plsc.subcore_barrier()
        pltpu.sync_copy(shared_mx, allmx_v)
        mxall = allmx_v[pl.ds(0, nl)]
        for i in range(1, ns):
            mxall = jnp.maximum(mxall, allmx_v[pl.ds(i * nl, nl)])
        m = jnp.max(mxall)
        for i in range(bw // nl):
            c = degp_v[pl.ds(off + i * nl, nl)]
            dn_v[pl.ds(i * nl, nl)] = jnp.where(m > 0, c / m, c)
        pltpu.sync_copy(dn_v, dn_out.at[pl.ds(base, bw)])

    return sc_degree


def kernel(env, indices, A, id_emb_weight):
    m, _ = A.shape
    b = indices.shape[0]
    d = id_emb_weight.shape[1]
    idx32 = indices.astype(jnp.int32)
    emb = _make_sc_gather(b, d)(id_emb_weight, idx32)
    m_tc = m - _SC_ROWS
    deg_sc = _make_sc_rowsum(m, _SC_ROWS, m_tc)(A)
    deg_tc = pl.pallas_call(
        _rowsum_body,
        grid=(m_tc // _ROWSUM_BM,),
        in_specs=[pl.BlockSpec((_ROWSUM_BM, m), lambda i: (i, 0))],
        out_specs=pl.BlockSpec((_ROWSUM_BM, 1), lambda i: (i, 0)),
        out_shape=jax.ShapeDtypeStruct((m_tc, 1), jnp.float32),
    )(A)
    deg = jnp.concatenate([deg_tc.reshape(m_tc), deg_sc])
    dn = _make_sc_degree(b)(idx32, deg)
    return jnp.concatenate(
        [emb, dn[:, None], jnp.zeros((b, 2), jnp.float32)], axis=1
    )


# speedup vs baseline: 3.0422x; 3.0422x over previous
import functools

import jax
import jax.numpy as jnp
from jax import lax
from jax.experimental import pallas as pl
from jax.experimental.pallas import tpu as pltpu
from jax.experimental.pallas import tpu_sc as plsc

_ROWSUM_BM = 256


def _rowsum_body(a_ref, o_ref):
    o_ref[...] = jnp.sum(a_ref[...], axis=1, keepdims=True)


@functools.lru_cache(maxsize=None)
def _make_sc_gather(b, d):
    info = plsc.get_sparse_core_info()
    nc, ns = info.num_cores, info.num_subcores
    nw = nc * ns
    bw = b // nw
    mesh = plsc.VectorSubcoreMesh(core_axis_name="c", subcore_axis_name="s")

    @functools.partial(
        pl.kernel,
        mesh=mesh,
        out_type=jax.ShapeDtypeStruct((b, d), jnp.float32),
        scratch_types=[
            pltpu.VMEM((bw,), jnp.int32),
            pltpu.VMEM((bw, d), jnp.float32),
            pltpu.SemaphoreType.DMA,
        ],
        compiler_params=pltpu.CompilerParams(
            use_tc_tiling_on_sc=False, needs_layout_passes=False
        ),
    )
    def sc_gather(table_hbm, idx_hbm, emb_out, idx_v, rows_v, sem):
        cid = lax.axis_index("c")
        sid = lax.axis_index("s")
        wid = sid * nc + cid
        base = wid * bw
        pltpu.sync_copy(idx_hbm.at[pl.ds(base, bw)], idx_v)
        cp = pltpu.async_copy(table_hbm.at[idx_v], rows_v, sem)
        cp.wait()
        pltpu.sync_copy(rows_v, emb_out.at[pl.ds(base, bw)])

    return sc_gather


@functools.lru_cache(maxsize=None)
def _make_sc_degree(b):
    info = plsc.get_sparse_core_info()
    nc, ns, nl = info.num_cores, info.num_subcores, info.num_lanes
    nw = nc * ns
    bw = b // nw
    bp = 2 * bw
    mesh = plsc.VectorSubcoreMesh(core_axis_name="c", subcore_axis_name="s")

    @functools.partial(
        pl.kernel,
        mesh=mesh,
        out_type=jax.ShapeDtypeStruct((b,), jnp.float32),
        scratch_types=[
            pltpu.VMEM((bp,), jnp.int32),
            pltpu.VMEM((bp,), jnp.float32),
            pltpu.VMEM((bw,), jnp.float32),
            pltpu.VMEM((nl,), jnp.float32),
            pltpu.VMEM((ns * nl,), jnp.float32),
            pltpu.VMEM_SHARED((ns * nl,), jnp.float32),
            pltpu.SemaphoreType.DMA,
        ],
        compiler_params=pltpu.CompilerParams(
            use_tc_tiling_on_sc=False, needs_layout_passes=False
        ),
    )
    def sc_degree(idx_hbm, deg_hbm, dn_out,
                  idxp_v, degp_v, dn_v, mx_v, allmx_v, shared_mx, sem):
        cid = lax.axis_index("c")
        sid = lax.axis_index("s")
        wid = sid * nc + cid
        base = wid * bw
        pair_base = (wid // 2) * bp
        off = (wid % 2) * bw
        pltpu.sync_copy(idx_hbm.at[pl.ds(pair_base, bp)], idxp_v)
        cp = pltpu.async_copy(deg_hbm.at[idxp_v], degp_v, sem)
        cp.wait()
        mx = degp_v[pl.ds(0, nl)]
        for i in range(1, bp // nl):
            mx = jnp.maximum(mx, degp_v[pl.ds(i * nl, nl)])
        mx_v[...] = mx
        pltpu.sync_copy(mx_v, shared_mx.at[pl.ds(sid * nl, nl)])
        plsc.subcore_barrier()
        pltpu.sync_copy(shared_mx, allmx_v)
        mxall = allmx_v[pl.ds(0, nl)]
        for i in range(1, ns):
            mxall = jnp.maximum(mxall, allmx_v[pl.ds(i * nl, nl)])
        m = jnp.max(mxall)
        for i in range(bw // nl):
            c = degp_v[pl.ds(off + i * nl, nl)]
            dn_v[pl.ds(i * nl, nl)] = jnp.where(m > 0, c / m, c)
        pltpu.sync_copy(dn_v, dn_out.at[pl.ds(base, bw)])

    return sc_degree


def kernel(env, indices, A, id_emb_weight):
    m, _ = A.shape
    b = indices.shape[0]
    d = id_emb_weight.shape[1]
    idx32 = indices.astype(jnp.int32)
    emb = _make_sc_gather(b, d)(id_emb_weight, idx32)
    deg2 = pl.pallas_call(
        _rowsum_body,
        grid=(m // _ROWSUM_BM,),
        in_specs=[pl.BlockSpec((_ROWSUM_BM, m), lambda i: (i, 0))],
        out_specs=pl.BlockSpec((_ROWSUM_BM, 1), lambda i: (i, 0)),
        out_shape=jax.ShapeDtypeStruct((m, 1), jnp.float32),
    )(A)
    dn = _make_sc_degree(b)(idx32, deg2.reshape(m))
    return jnp.concatenate(
        [emb, dn[:, None], jnp.zeros((b, 2), jnp.float32)], axis=1
    )
